# Initial kernel scaffold; baseline (speedup 1.0000x reference)
import jax
import jax.numpy as jnp
from jax.experimental import pallas as pl


def kernel(x, position_weight, value_weight, classify_weight):
    raise NotImplementedError("write your pallas kernel here")


# speedup vs baseline: 69.2368x; 69.2368x over previous
import jax
import jax.numpy as jnp
from jax.experimental import pallas as pl


def _fused_kernel(x_ref, pos_ref, vw_ref, cw_ref, out_ref):
    t = (x_ref[...] > 0.5).astype(jnp.float32)
    pos = pos_ref[...]
    a = jnp.dot(t, pos, preferred_element_type=jnp.float32)
    p = jnp.sum(pos, axis=0, keepdims=True)
    v0 = vw_ref[0:1, :]
    v1 = vw_ref[1:2, :]
    sample = v0 * (p - a) + v1 * a
    enc = jnp.where(sample > 0, 1.0, -1.0)
    out_ref[...] = jnp.dot(enc, cw_ref[...].T,
                           preferred_element_type=jnp.float32)


def kernel(x, position_weight, value_weight, classify_weight):
    B = x.shape[0]
    S = x.shape[1] * x.shape[2]
    x_flat = x.reshape(B, S)
    return pl.pallas_call(
        _fused_kernel,
        out_shape=jax.ShapeDtypeStruct((B, classify_weight.shape[0]),
                                       jnp.float32),
    )(x_flat, position_weight, value_weight, classify_weight)
